# baseline (speedup 1.0000x reference)
import functools

import jax
import jax.numpy as jnp
from jax import lax
from jax.experimental import pallas as pl
from jax.experimental.pallas import tpu as pltpu
from jax.experimental.pallas import tpu_sc as plsc

B = 16384
F = 32
NC, NS, L = 2, 16, 16
NW = NC * NS
BPW = B // NW
CHUNK = 128
NCHUNK = BPW // CHUNK


def _mf_body(user_hbm, item_hbm, uf_hbm, if_hbm, out_hbm,
             uidx, iidx, urows, vrows, outv, sem):
    wid = lax.axis_index("s") * NC + lax.axis_index("c")
    base = wid * BPW

    pltpu.sync_copy(user_hbm.at[pl.ds(base, BPW)], uidx)
    pltpu.sync_copy(item_hbm.at[pl.ds(base, BPW)], iidx)

    copies = []
    for c in range(NCHUNK):
        sl = pl.ds(c * CHUNK, CHUNK)
        copies.append(pltpu.async_copy(uf_hbm.at[uidx.at[sl]], urows.at[sl], sem))
        copies.append(pltpu.async_copy(if_hbm.at[iidx.at[sl]], vrows.at[sl], sem))
    for cp in copies:
        cp.wait()

    def blk_body(blk, carry):
        rows = blk * L + lax.iota(jnp.int32, L)
        acc = jnp.zeros((L,), jnp.float32)
        for f in range(F):
            col = jnp.full((L,), f, jnp.int32)
            uu = plsc.load_gather(urows, [rows, col])
            vv = plsc.load_gather(vrows, [rows, col])
            acc = acc + uu * vv
        outv[pl.ds(blk * L, L)] = acc
        return carry

    lax.fori_loop(0, BPW // L, blk_body, 0)

    pltpu.sync_copy(outv, out_hbm.at[pl.ds(base, BPW)])


_mf = functools.partial(
    pl.kernel,
    out_type=jax.ShapeDtypeStruct((B,), jnp.float32),
    mesh=plsc.VectorSubcoreMesh(core_axis_name="c", subcore_axis_name="s",
                                num_cores=NC, num_subcores=NS),
    compiler_params=pltpu.CompilerParams(needs_layout_passes=False,
                                         use_tc_tiling_on_sc=False),
    scratch_types=[
        pltpu.VMEM((BPW,), jnp.int32),
        pltpu.VMEM((BPW,), jnp.int32),
        pltpu.VMEM((BPW, F), jnp.float32),
        pltpu.VMEM((BPW, F), jnp.float32),
        pltpu.VMEM((BPW,), jnp.float32),
        pltpu.SemaphoreType.DMA,
    ],
)(_mf_body)


def kernel(user, item, user_factors, item_factors):
    return _mf(user.astype(jnp.int32), item.astype(jnp.int32),
               user_factors, item_factors)


# speedup vs baseline: 1.4848x; 1.4848x over previous
import functools

import jax
import jax.numpy as jnp
from jax import lax
from jax.experimental import pallas as pl
from jax.experimental.pallas import tpu as pltpu
from jax.experimental.pallas import tpu_sc as plsc

B = 16384
F = 32
NC, NS, L = 2, 16, 16
NW = NC * NS
BPW = B // NW
CH = 256
NPASS = BPW // CH


def _mf_body(user_hbm, item_hbm, uf_hbm, if_hbm, out_hbm,
             uidx_v, iidx_v, urows, vrows, outv, sem):
    wid = lax.axis_index("s") * NC + lax.axis_index("c")
    base = wid * BPW

    pltpu.sync_copy(user_hbm.at[pl.ds(base, BPW)], uidx_v)
    pltpu.sync_copy(item_hbm.at[pl.ds(base, BPW)], iidx_v)

    def one_pass(p, carry):
        def fire(j, c):
            uvec = uidx_v[pl.ds(p * CH + j * L, L)]
            ivec = iidx_v[pl.ds(p * CH + j * L, L)]
            for k in range(L):
                pltpu.async_copy(uf_hbm.at[uvec[k]], urows.at[j * L + k], sem)
                pltpu.async_copy(if_hbm.at[ivec[k]], vrows.at[j * L + k], sem)
            return c

        lax.fori_loop(0, CH // L, fire, 0)

        pltpu.make_async_copy(uf_hbm.at[pl.ds(0, CH), :], urows, sem).wait()
        pltpu.make_async_copy(uf_hbm.at[pl.ds(0, CH), :], vrows, sem).wait()

        def blk_body(blk, c):
            rows = blk * L + lax.iota(jnp.int32, L)
            acc = jnp.zeros((L,), jnp.float32)
            for f in range(F):
                col = jnp.full((L,), f, jnp.int32)
                uu = plsc.load_gather(urows, [rows, col])
                vv = plsc.load_gather(vrows, [rows, col])
                acc = acc + uu * vv
            outv[pl.ds(p * CH + blk * L, L)] = acc
            return c

        lax.fori_loop(0, CH // L, blk_body, 0)
        return carry

    lax.fori_loop(0, NPASS, one_pass, 0)

    pltpu.sync_copy(outv, out_hbm.at[pl.ds(base, BPW)])


_mf = functools.partial(
    pl.kernel,
    out_type=jax.ShapeDtypeStruct((B,), jnp.float32),
    mesh=plsc.VectorSubcoreMesh(core_axis_name="c", subcore_axis_name="s",
                                num_cores=NC, num_subcores=NS),
    compiler_params=pltpu.CompilerParams(needs_layout_passes=False),
    scratch_types=[
        pltpu.VMEM((BPW,), jnp.int32),
        pltpu.VMEM((BPW,), jnp.int32),
        pltpu.VMEM((CH, F), jnp.float32),
        pltpu.VMEM((CH, F), jnp.float32),
        pltpu.VMEM((BPW,), jnp.float32),
        pltpu.SemaphoreType.DMA,
    ],
)(_mf_body)


def kernel(user, item, user_factors, item_factors):
    return _mf(user.astype(jnp.int32), item.astype(jnp.int32),
               user_factors, item_factors)


# speedup vs baseline: 3.6099x; 2.4313x over previous
import functools

import jax
import jax.numpy as jnp
from jax import lax
from jax.experimental import pallas as pl
from jax.experimental.pallas import tpu as pltpu
from jax.experimental.pallas import tpu_sc as plsc

B = 16384
F = 32
NC, NS, L = 2, 16, 16
NW = NC * NS
BPW = B // NW
W = 128
K = 8


def _mf_body(user_hbm, item_hbm, uft_hbm, ift_hbm, out_hbm,
             uidx_v, iidx_v, uwins, vwins, outv, sem):
    wid = lax.axis_index("s") * NC + lax.axis_index("c")
    base = wid * BPW

    pltpu.sync_copy(user_hbm.at[pl.ds(base, BPW)], uidx_v)
    pltpu.sync_copy(item_hbm.at[pl.ds(base, BPW)], iidx_v)

    iota = lax.iota(jnp.int32, L)

    def step(j, carry):
        uvec = uidx_v[pl.ds(j * L, L)]
        ivec = iidx_v[pl.ds(j * L, L)]
        uh = (uvec // W) * W
        ih = (ivec // W) * W
        uq = uvec - uh
        iq = ivec - ih
        res = jnp.zeros((L,), jnp.float32)
        for half in range(L // K):
            copies = []
            for k in range(K):
                lane = half * K + k
                copies.append(pltpu.async_copy(
                    uft_hbm.at[:, pl.ds(pl.multiple_of(uh[lane], W), W)],
                    uwins.at[k], sem))
                copies.append(pltpu.async_copy(
                    ift_hbm.at[:, pl.ds(pl.multiple_of(ih[lane], W), W)],
                    vwins.at[k], sem))
            for cp in copies:
                cp.wait()
            for k in range(K):
                lane = half * K + k
                slot = jnp.full((L,), k, jnp.int32)
                uql = jnp.full((L,), uq[lane], jnp.int32)
                iql = jnp.full((L,), iq[lane], jnp.int32)
                ulo = plsc.load_gather(uwins, [slot, iota, uql])
                uhi = plsc.load_gather(uwins, [slot, iota + L, uql])
                vlo = plsc.load_gather(vwins, [slot, iota, iql])
                vhi = plsc.load_gather(vwins, [slot, iota + L, iql])
                dot = jnp.sum(ulo * vlo + uhi * vhi, axis=0)
                res = jnp.where(iota == lane, dot, res)
        outv[pl.ds(j * L, L)] = res
        return carry

    lax.fori_loop(0, BPW // L, step, 0)

    pltpu.sync_copy(outv, out_hbm.at[pl.ds(base, BPW)])


_mf = functools.partial(
    pl.kernel,
    out_type=jax.ShapeDtypeStruct((B,), jnp.float32),
    mesh=plsc.VectorSubcoreMesh(core_axis_name="c", subcore_axis_name="s",
                                num_cores=NC, num_subcores=NS),
    compiler_params=pltpu.CompilerParams(needs_layout_passes=False),
    scratch_types=[
        pltpu.VMEM((BPW,), jnp.int32),
        pltpu.VMEM((BPW,), jnp.int32),
        pltpu.VMEM((K, F, W), jnp.float32),
        pltpu.VMEM((K, F, W), jnp.float32),
        pltpu.VMEM((BPW,), jnp.float32),
        pltpu.SemaphoreType.DMA,
    ],
)(_mf_body)


def kernel(user, item, user_factors, item_factors):
    return _mf(user.astype(jnp.int32), item.astype(jnp.int32),
               user_factors.T, item_factors.T)
